# baseline (speedup 1.0000x reference)
import functools

import jax
import jax.numpy as jnp
from jax import lax
from jax.experimental import pallas as pl
from jax.experimental.pallas import tpu as pltpu
from jax.experimental.pallas import tpu_sc as plsc

_L = 128
_NC = 2
_NT = 16


def _elu(v):
    return jnp.where(v > 0, v, jnp.exp(jnp.minimum(v, 0.0)) - 1.0)


def _layernorm(h, w, b):
    mu = jnp.mean(h, axis=-1, keepdims=True)
    var = jnp.mean((h - mu) ** 2, axis=-1, keepdims=True)
    return (h - mu) * lax.rsqrt(var + 1e-5) * w + b


def _softmax(z):
    z = z - jnp.max(z, axis=-1, keepdims=True)
    e = jnp.exp(z)
    return e / jnp.sum(e, axis=-1, keepdims=True)


@functools.lru_cache(maxsize=None)
def _make_sc_scatter(n_rel, n_tab, w, n, n_grp):
    mesh = plsc.VectorSubcoreMesh(core_axis_name="c", subcore_axis_name="s")
    nw = _NC * _NT
    rpt = n // _NT
    nfull = n_grp // nw
    extra = n_grp % nw
    K = next((k for k in (8, 7, 6, 5, 4) if nfull % k == 0), 1)
    n_ch = nfull // K
    out_t = jax.ShapeDtypeStruct((n_rel, _NC, n, w), jnp.float32)
    scratch = (
        [pltpu.VMEM((nfull + 1, _L), jnp.int32),
         pltpu.VMEM((nfull + 1, _L), jnp.int32),
         pltpu.VMEM((2, K, _L, w), jnp.float32),
         pltpu.VMEM_SHARED((n, w), jnp.float32),
         pltpu.VMEM_SHARED((n, w), jnp.float32)]
        + [pltpu.SemaphoreType.DMA] * (4 * K)
    )

    @functools.partial(
        pl.kernel, mesh=mesh, out_type=out_t, scratch_types=scratch,
        compiler_params=pltpu.CompilerParams(use_tc_tiling_on_sc=False))
    def k(*refs):
        tabs = refs[:n_tab]
        eis = refs[n_tab:n_tab + n_rel]
        zb = refs[n_tab + n_rel]
        out = refs[n_tab + n_rel + 1]
        src_buf, dst_buf, rows, acc, tab_s = refs[-(5 + 4 * K):-4 * K]
        sems = refs[-4 * K:]
        gsems = [sems[:K], sems[K:2 * K]]
        ssems = [sems[2 * K:3 * K], sems[3 * K:]]

        c = lax.axis_index("c")
        s = lax.axis_index("s")
        wid = c * _NT + s
        base = wid * nfull + jnp.minimum(wid, extra)
        has_extra = wid < extra
        row0 = s * rpt

        pltpu.sync_copy(zb, acc.at[pl.ds(row0, rpt)])
        pltpu.sync_copy(tabs[0].at[pl.ds(row0, rpt)],
                        tab_s.at[pl.ds(row0, rpt)])
        plsc.subcore_barrier()

        for r in range(n_rel):
            ei = eis[r]

            @pl.when(has_extra)
            def _():
                pltpu.sync_copy(ei.at[0, pl.ds(base, nfull + 1)], src_buf)
                pltpu.sync_copy(ei.at[1, pl.ds(base, nfull + 1)], dst_buf)

            @pl.when(jnp.logical_not(has_extra))
            def _():
                pltpu.sync_copy(ei.at[0, pl.ds(base, nfull)],
                                src_buf.at[pl.ds(0, nfull)])
                pltpu.sync_copy(ei.at[1, pl.ds(base, nfull)],
                                dst_buf.at[pl.ds(0, nfull)])

            for slot in range(K):
                pltpu.async_copy(tab_s.at[src_buf.at[slot]],
                                 rows.at[0, slot], gsems[0][slot])

            def half(ch, p, q, first):
                for slot in range(K):
                    g = ch * K + slot
                    pltpu.make_async_copy(tab_s.at[src_buf.at[g]],
                                          rows.at[p, slot],
                                          gsems[p][slot]).wait()
                    pltpu.async_copy(rows.at[p, slot],
                                     acc.at[dst_buf.at[g]],
                                     ssems[p][slot], add=True)
                for slot in range(K):
                    gp = (ch - 1) * K + slot

                    def wait_prev(slot=slot, gp=gp):
                        pltpu.make_async_copy(rows.at[q, slot],
                                              acc.at[dst_buf.at[gp]],
                                              ssems[q][slot]).wait()

                    if first:
                        pl.when(ch >= 1)(wait_prev)
                    else:
                        wait_prev()

                    @pl.when(ch < n_ch - 1)
                    def _(slot=slot):
                        g2 = (ch + 1) * K + slot
                        pltpu.async_copy(tab_s.at[src_buf.at[g2]],
                                         rows.at[q, slot], gsems[q][slot])

            def pair(i, carry):
                half(2 * i, 0, 1, True)
                half(2 * i + 1, 1, 0, False)
                return carry

            n_pair = n_ch // 2
            lax.fori_loop(0, n_pair, pair, 0)

            last = n_ch - 1
            lastp = last % 2
            if n_ch % 2 == 1:
                ch = last
                for slot in range(K):
                    g = ch * K + slot
                    pltpu.make_async_copy(tab_s.at[src_buf.at[g]],
                                          rows.at[lastp, slot],
                                          gsems[lastp][slot]).wait()
                    pltpu.async_copy(rows.at[lastp, slot],
                                     acc.at[dst_buf.at[g]],
                                     ssems[lastp][slot], add=True)
                for slot in range(K):
                    gp = (ch - 1) * K + slot
                    pltpu.make_async_copy(rows.at[1 - lastp, slot],
                                          acc.at[dst_buf.at[gp]],
                                          ssems[1 - lastp][slot]).wait()
            for slot in range(K):
                g = last * K + slot
                pltpu.make_async_copy(rows.at[lastp, slot],
                                      acc.at[dst_buf.at[g]],
                                      ssems[lastp][slot]).wait()

            @pl.when(has_extra)
            def _():
                pltpu.async_copy(tab_s.at[src_buf.at[nfull]],
                                 rows.at[0, 0], gsems[0][0]).wait()
                pltpu.sync_copy(rows.at[0, 0], acc.at[dst_buf.at[nfull]],
                                add=True)

            plsc.subcore_barrier()
            pltpu.sync_copy(acc.at[pl.ds(row0, rpt)],
                            out.at[r, c, pl.ds(row0, rpt)])
            if r < n_rel - 1:
                pltpu.sync_copy(zb, acc.at[pl.ds(row0, rpt)])
                if n_tab > 1:
                    pltpu.sync_copy(tabs[r + 1].at[pl.ds(row0, rpt)],
                                    tab_s.at[pl.ds(row0, rpt)])
                plsc.subcore_barrier()

    return k


def kernel(x, contig_ei, alliance_ei, trade_ei, W_contig, b_contig,
           W_alliance, b_alliance, W_trade, b_trade, ln1_w, ln1_b,
           W_conv2, b_conv2, ln2_w, ln2_b, Wl1, bl1, Wl2, bl2,
           Wf1, bf1, Wf2, bf2, Wh1, bh1, Wh2, bh2):
    N, D = x.shape
    H = W_contig.shape[1]
    T = Wh1.shape[0] - H
    O = Wh2.shape[1]
    E = contig_ei.shape[1]
    n_grp = E // _L
    rpt = N // _NT
    wa = -(-(T + 1) // 8) * 8

    f32 = jnp.float32
    ei_c = contig_ei.astype(jnp.int32).reshape(2, n_grp, _L)
    ei_a = alliance_ei.astype(jnp.int32).reshape(2, n_grp, _L)
    ei_t = trade_ei.astype(jnp.int32).reshape(2, n_grp, _L)

    tab_a = jnp.concatenate(
        [x[:, :T], jnp.ones((N, 1), f32), jnp.zeros((N, wa - T - 1), f32)],
        axis=1)
    zb_a = jnp.zeros((rpt, wa), f32)
    zb_h = jnp.zeros((rpt, H), f32)

    r2 = lambda v: v.reshape(1, -1)

    n_blk = 10
    rb = N // n_blk

    def rspec(*shape):
        nlead = len(shape) - 2
        return pl.BlockSpec(
            shape[:nlead] + (rb, shape[-1]),
            lambda i, nlead=nlead: (0,) * nlead + (i, 0))

    def fspec(*shape):
        return pl.BlockSpec(shape, lambda i, n=len(shape): (0,) * n)

    def tc1(x_ref, wc, wa_, wt, xwc_o, xwa_o, xwt_o, t_o):
        xv = x_ref[...]
        dot = lambda a, b: jnp.dot(a, b, preferred_element_type=f32)
        xwc_o[...] = dot(xv, wc[...])
        xwa_o[...] = dot(xv, wa_[...])
        xwt_o[...] = dot(xv, wt[...])
        t_o[...] = xv[:, :T]

    xw_c, xw_a, xw_t, t_arr = pl.pallas_call(
        tc1,
        grid=(n_blk,),
        in_specs=[rspec(N, D), fspec(D, H), fspec(D, H), fspec(D, H)],
        out_specs=[rspec(N, H)] * 3 + [rspec(N, T)],
        out_shape=[jax.ShapeDtypeStruct((N, H), f32)] * 3
        + [jax.ShapeDtypeStruct((N, T), f32)],
    )(x, W_contig, W_alliance, W_trade)

    def tc1b(x_ref, wl1, bl1_, wl2, bl2_, wh1h, wh1t, bh1_, wh2, bh2_,
             yl_o):
        xv = x_ref[...]
        dot = lambda a, b: jnp.dot(a, b, preferred_element_type=f32)
        hl = _elu(dot(xv, wl1[...]) + bl1_[...])
        hl = _elu(dot(hl, wl2[...]) + bl2_[...])
        z = _elu(dot(hl, wh1h[...]) + dot(xv[:, :T], wh1t[...]) + bh1_[...])
        yl_o[...] = _softmax(dot(z, wh2[...]) + bh2_[...])

    y_local = pl.pallas_call(
        tc1b,
        grid=(n_blk,),
        in_specs=[rspec(N, D), fspec(D, H), fspec(1, H), fspec(H, H),
                  fspec(1, H), fspec(H, H), fspec(T, H), fspec(1, H),
                  fspec(H, O), fspec(1, O)],
        out_specs=rspec(N, O),
        out_shape=jax.ShapeDtypeStruct((N, O), f32),
    )(x, Wl1, r2(bl1), Wl2, r2(bl2), Wh1[:H], Wh1[H:], r2(bh1), Wh2,
      r2(bh2))

    deg_p = _make_sc_scatter(3, 1, wa, N, n_grp)(
        tab_a, ei_c, ei_a, ei_t, zb_a)

    def tc2(dp_ref, xwc_ref, xwa_ref, xwt_ref, expo_o, dis_o, uc_o, ua_o,
            ut_o):
        dp = dp_ref[...]
        tot = dp[:, 0] + dp[:, 1]
        indeg = tot[:, :, T:T + 1]
        num = tot[:, :, :T]
        expo = num / jnp.maximum(indeg, 1.0)
        expo_o[...] = jnp.concatenate([expo[0], expo[1], expo[2]], axis=-1)
        dis = lax.rsqrt(indeg + 1.0)
        dis_o[...] = dis
        uc_o[...] = xwc_ref[...] * dis[0]
        ua_o[...] = xwa_ref[...] * dis[1]
        ut_o[...] = xwt_ref[...] * dis[2]

    expo, dis, u_c, u_a, u_t = pl.pallas_call(
        tc2,
        grid=(n_blk,),
        in_specs=[rspec(3, 2, N, wa)] + [rspec(N, H)] * 3,
        out_specs=[rspec(N, 3 * T), rspec(3, N, 1)] + [rspec(N, H)] * 3,
        out_shape=[jax.ShapeDtypeStruct((N, 3 * T), f32),
                   jax.ShapeDtypeStruct((3, N, 1), f32)]
        + [jax.ShapeDtypeStruct((N, H), f32)] * 3,
    )(deg_p, xw_c, xw_a, xw_t)

    seg_p = _make_sc_scatter(3, 3, H, N, n_grp)(
        u_c, u_a, u_t, ei_c, ei_a, ei_t, zb_h)

    def tc3(sp_ref, uc_ref, ua_ref, ut_ref, dis_ref, bc, ba, bt, l1w, l1b,
            w2, u2_o):
        sp = sp_ref[...]
        S = sp[:, 0] + sp[:, 1]
        dis = dis_ref[...]
        h = (dis[0] * (S[0] + uc_ref[...]) + bc[...]
             + dis[1] * (S[1] + ua_ref[...]) + ba[...]
             + dis[2] * (S[2] + ut_ref[...]) + bt[...])
        h = _elu(_layernorm(h, l1w[...], l1b[...]))
        u2_o[...] = jnp.dot(h, w2[...], preferred_element_type=f32) * dis[0]

    u2 = pl.pallas_call(
        tc3,
        grid=(n_blk,),
        in_specs=[rspec(3, 2, N, H)] + [rspec(N, H)] * 3
        + [rspec(3, N, 1)] + [fspec(1, H)] * 5 + [fspec(H, H)],
        out_specs=rspec(N, H),
        out_shape=jax.ShapeDtypeStruct((N, H), f32),
    )(seg_p, u_c, u_a, u_t, dis, r2(b_contig), r2(b_alliance), r2(b_trade),
      r2(ln1_w), r2(ln1_b), W_conv2)

    seg2_p = _make_sc_scatter(1, 1, H, N, n_grp)(u2, ei_c, zb_h)

    def tc4(sp_ref, u2_ref, dis_ref, b2, l2w, l2b, t_ref, expo_ref, wf1h,
            wf1t, wf1e, bf1_, wf2, bf2_, yf_o):
        sp = sp_ref[...]
        S2 = sp[0, 0] + sp[0, 1]
        dis = dis_ref[...]
        g = dis[0] * (S2 + u2_ref[...]) + b2[...]
        ho = _elu(_layernorm(g, l2w[...], l2b[...]))
        dot = lambda a, b: jnp.dot(a, b, preferred_element_type=f32)
        z = _elu(dot(ho, wf1h[...]) + dot(t_ref[...], wf1t[...])
                 + dot(expo_ref[...], wf1e[...]) + bf1_[...])
        yf_o[...] = _softmax(dot(z, wf2[...]) + bf2_[...])

    y_full = pl.pallas_call(
        tc4,
        grid=(n_blk,),
        in_specs=[rspec(1, 2, N, H), rspec(N, H), rspec(3, N, 1),
                  fspec(1, H), fspec(1, H), fspec(1, H),
                  rspec(N, T),
                  rspec(N, 3 * T), fspec(H, H), fspec(T, H),
                  fspec(3 * T, H), fspec(1, H), fspec(H, O), fspec(1, O)],
        out_specs=rspec(N, O),
        out_shape=jax.ShapeDtypeStruct((N, O), f32),
    )(seg2_p, u2, dis, r2(b_conv2), r2(ln2_w), r2(ln2_b), t_arr, expo,
      Wf1[:H], Wf1[H:H + T], Wf1[H + T:], r2(bf1), Wf2, r2(bf2))

    return (y_full, y_local, expo)


# speedup vs baseline: 1.0071x; 1.0071x over previous
import functools

import jax
import jax.numpy as jnp
from jax import lax
from jax.experimental import pallas as pl
from jax.experimental.pallas import tpu as pltpu
from jax.experimental.pallas import tpu_sc as plsc

_L = 128
_NC = 2
_NT = 16


def _elu(v):
    return jnp.where(v > 0, v, jnp.exp(jnp.minimum(v, 0.0)) - 1.0)


def _layernorm(h, w, b):
    mu = jnp.mean(h, axis=-1, keepdims=True)
    var = jnp.mean((h - mu) ** 2, axis=-1, keepdims=True)
    return (h - mu) * lax.rsqrt(var + 1e-5) * w + b


def _softmax(z):
    z = z - jnp.max(z, axis=-1, keepdims=True)
    e = jnp.exp(z)
    return e / jnp.sum(e, axis=-1, keepdims=True)


@functools.lru_cache(maxsize=None)
def _make_sc_scatter(n_rel, n_tab, w, n, n_grp):
    mesh = plsc.VectorSubcoreMesh(core_axis_name="c", subcore_axis_name="s")
    nw = _NC * _NT
    rpt = n // _NT
    nfull = n_grp // nw
    extra = n_grp % nw
    K = next((k for k in (8, 7, 6, 5, 4) if nfull % k == 0), 1)
    n_ch = nfull // K
    out_t = jax.ShapeDtypeStruct((n_rel, _NC, n, w), jnp.float32)
    scratch = (
        [pltpu.VMEM((nfull + 1, _L), jnp.int32),
         pltpu.VMEM((nfull + 1, _L), jnp.int32),
         pltpu.VMEM((2, K, _L, w), jnp.float32),
         pltpu.VMEM_SHARED((n, w), jnp.float32),
         pltpu.VMEM_SHARED((n, w), jnp.float32)]
        + [pltpu.SemaphoreType.DMA] * (4 * K)
    )

    @functools.partial(
        pl.kernel, mesh=mesh, out_type=out_t, scratch_types=scratch,
        compiler_params=pltpu.CompilerParams(use_tc_tiling_on_sc=False))
    def k(*refs):
        tabs = refs[:n_tab]
        eis = refs[n_tab:n_tab + n_rel]
        zb = refs[n_tab + n_rel]
        out = refs[n_tab + n_rel + 1]
        src_buf, dst_buf, rows, acc, tab_s = refs[-(5 + 4 * K):-4 * K]
        sems = refs[-4 * K:]
        gsems = [sems[:K], sems[K:2 * K]]
        ssems = [sems[2 * K:3 * K], sems[3 * K:]]

        c = lax.axis_index("c")
        s = lax.axis_index("s")
        wid = c * _NT + s
        base = wid * nfull + jnp.minimum(wid, extra)
        has_extra = wid < extra
        row0 = s * rpt

        pltpu.sync_copy(zb, acc.at[pl.ds(row0, rpt)])
        pltpu.sync_copy(tabs[0].at[pl.ds(row0, rpt)],
                        tab_s.at[pl.ds(row0, rpt)])
        plsc.subcore_barrier()

        for r in range(n_rel):
            ei = eis[r]

            @pl.when(has_extra)
            def _():
                pltpu.sync_copy(ei.at[0, pl.ds(base, nfull + 1)], src_buf)
                pltpu.sync_copy(ei.at[1, pl.ds(base, nfull + 1)], dst_buf)

            @pl.when(jnp.logical_not(has_extra))
            def _():
                pltpu.sync_copy(ei.at[0, pl.ds(base, nfull)],
                                src_buf.at[pl.ds(0, nfull)])
                pltpu.sync_copy(ei.at[1, pl.ds(base, nfull)],
                                dst_buf.at[pl.ds(0, nfull)])

            for slot in range(K):
                pltpu.async_copy(tab_s.at[src_buf.at[slot]],
                                 rows.at[0, slot], gsems[0][slot])

            def half(ch, p, q, first):
                for slot in range(K):
                    g = ch * K + slot
                    pltpu.make_async_copy(tab_s.at[src_buf.at[g]],
                                          rows.at[p, slot],
                                          gsems[p][slot]).wait()
                    pltpu.async_copy(rows.at[p, slot],
                                     acc.at[dst_buf.at[g]],
                                     ssems[p][slot], add=True)
                for slot in range(K):
                    gp = (ch - 1) * K + slot

                    def wait_prev(slot=slot, gp=gp):
                        pltpu.make_async_copy(rows.at[q, slot],
                                              acc.at[dst_buf.at[gp]],
                                              ssems[q][slot]).wait()

                    if first:
                        pl.when(ch >= 1)(wait_prev)
                    else:
                        wait_prev()

                    @pl.when(ch < n_ch - 1)
                    def _(slot=slot):
                        g2 = (ch + 1) * K + slot
                        pltpu.async_copy(tab_s.at[src_buf.at[g2]],
                                         rows.at[q, slot], gsems[q][slot])

            def pair(i, carry):
                half(2 * i, 0, 1, True)
                half(2 * i + 1, 1, 0, False)
                return carry

            n_pair = n_ch // 2
            lax.fori_loop(0, n_pair, pair, 0)

            last = n_ch - 1
            lastp = last % 2
            if n_ch % 2 == 1:
                ch = last
                for slot in range(K):
                    g = ch * K + slot
                    pltpu.make_async_copy(tab_s.at[src_buf.at[g]],
                                          rows.at[lastp, slot],
                                          gsems[lastp][slot]).wait()
                    pltpu.async_copy(rows.at[lastp, slot],
                                     acc.at[dst_buf.at[g]],
                                     ssems[lastp][slot], add=True)
                for slot in range(K):
                    gp = (ch - 1) * K + slot
                    pltpu.make_async_copy(rows.at[1 - lastp, slot],
                                          acc.at[dst_buf.at[gp]],
                                          ssems[1 - lastp][slot]).wait()
            for slot in range(K):
                g = last * K + slot
                pltpu.make_async_copy(rows.at[lastp, slot],
                                      acc.at[dst_buf.at[g]],
                                      ssems[lastp][slot]).wait()

            @pl.when(has_extra)
            def _():
                pltpu.async_copy(tab_s.at[src_buf.at[nfull]],
                                 rows.at[0, 0], gsems[0][0]).wait()
                pltpu.sync_copy(rows.at[0, 0], acc.at[dst_buf.at[nfull]],
                                add=True)

            plsc.subcore_barrier()
            pltpu.sync_copy(acc.at[pl.ds(row0, rpt)],
                            out.at[r, c, pl.ds(row0, rpt)])
            if r < n_rel - 1:
                pltpu.sync_copy(zb, acc.at[pl.ds(row0, rpt)])
                if n_tab > 1:
                    pltpu.sync_copy(tabs[r + 1].at[pl.ds(row0, rpt)],
                                    tab_s.at[pl.ds(row0, rpt)])
                plsc.subcore_barrier()

    return k


def kernel(x, contig_ei, alliance_ei, trade_ei, W_contig, b_contig,
           W_alliance, b_alliance, W_trade, b_trade, ln1_w, ln1_b,
           W_conv2, b_conv2, ln2_w, ln2_b, Wl1, bl1, Wl2, bl2,
           Wf1, bf1, Wf2, bf2, Wh1, bh1, Wh2, bh2):
    N, D = x.shape
    H = W_contig.shape[1]
    T = Wh1.shape[0] - H
    O = Wh2.shape[1]
    E = contig_ei.shape[1]
    n_grp = E // _L
    rpt = N // _NT
    wa = -(-(T + 1) // 8) * 8

    f32 = jnp.float32
    ei_c = contig_ei.astype(jnp.int32).reshape(2, n_grp, _L)
    ei_a = alliance_ei.astype(jnp.int32).reshape(2, n_grp, _L)
    ei_t = trade_ei.astype(jnp.int32).reshape(2, n_grp, _L)

    tab_a = jnp.concatenate(
        [x[:, :T], jnp.ones((N, 1), f32), jnp.zeros((N, wa - T - 1), f32)],
        axis=1)
    zb_a = jnp.zeros((rpt, wa), f32)
    zb_h = jnp.zeros((rpt, H), f32)

    r2 = lambda v: v.reshape(1, -1)

    n_blk = 10
    rb = N // n_blk

    def rspec(*shape):
        nlead = len(shape) - 2
        return pl.BlockSpec(
            shape[:nlead] + (rb, shape[-1]),
            lambda i, nlead=nlead: (0,) * nlead + (i, 0))

    def fspec(*shape):
        return pl.BlockSpec(shape, lambda i, n=len(shape): (0,) * n)

    def tc1(x_ref, wc, wa_, wt, wl1, bl1_, wl2, bl2_, wh1h, wh1t, bh1_,
            wh2, bh2_, xwc_o, xwa_o, xwt_o, t_o, yl_o):
        xv = x_ref[...]
        dot = lambda a, b: jnp.dot(a, b, preferred_element_type=f32)
        xwc_o[...] = dot(xv, wc[...])
        xwa_o[...] = dot(xv, wa_[...])
        xwt_o[...] = dot(xv, wt[...])
        hl = _elu(dot(xv, wl1[...]) + bl1_[...])
        hl = _elu(dot(hl, wl2[...]) + bl2_[...])
        tt = xv[:, :T]
        t_o[...] = tt
        z = _elu(dot(hl, wh1h[...]) + dot(tt, wh1t[...]) + bh1_[...])
        yl_o[...] = _softmax(dot(z, wh2[...]) + bh2_[...])

    xw_c, xw_a, xw_t, t_arr, y_local = pl.pallas_call(
        tc1,
        grid=(n_blk,),
        in_specs=[rspec(N, D), fspec(D, H), fspec(D, H), fspec(D, H),
                  fspec(D, H), fspec(1, H), fspec(H, H), fspec(1, H),
                  fspec(H, H), fspec(T, H), fspec(1, H), fspec(H, O),
                  fspec(1, O)],
        out_specs=[rspec(N, H)] * 3 + [rspec(N, T), rspec(N, O)],
        out_shape=[jax.ShapeDtypeStruct((N, H), f32)] * 3
        + [jax.ShapeDtypeStruct((N, T), f32),
           jax.ShapeDtypeStruct((N, O), f32)],
    )(x, W_contig, W_alliance, W_trade, Wl1, r2(bl1), Wl2, r2(bl2),
      Wh1[:H], Wh1[H:], r2(bh1), Wh2, r2(bh2))

    deg_p = _make_sc_scatter(3, 1, wa, N, n_grp)(
        tab_a, ei_c, ei_a, ei_t, zb_a)

    def tc2(dp_ref, xwc_ref, xwa_ref, xwt_ref, expo_o, dis_o, uc_o, ua_o,
            ut_o):
        dp = dp_ref[...]
        tot = dp[:, 0] + dp[:, 1]
        indeg = tot[:, :, T:T + 1]
        num = tot[:, :, :T]
        expo = num / jnp.maximum(indeg, 1.0)
        expo_o[...] = jnp.concatenate([expo[0], expo[1], expo[2]], axis=-1)
        dis = lax.rsqrt(indeg + 1.0)
        dis_o[...] = dis
        uc_o[...] = xwc_ref[...] * dis[0]
        ua_o[...] = xwa_ref[...] * dis[1]
        ut_o[...] = xwt_ref[...] * dis[2]

    expo, dis, u_c, u_a, u_t = pl.pallas_call(
        tc2,
        grid=(n_blk,),
        in_specs=[rspec(3, 2, N, wa)] + [rspec(N, H)] * 3,
        out_specs=[rspec(N, 3 * T), rspec(3, N, 1)] + [rspec(N, H)] * 3,
        out_shape=[jax.ShapeDtypeStruct((N, 3 * T), f32),
                   jax.ShapeDtypeStruct((3, N, 1), f32)]
        + [jax.ShapeDtypeStruct((N, H), f32)] * 3,
    )(deg_p, xw_c, xw_a, xw_t)

    seg_p = _make_sc_scatter(3, 3, H, N, n_grp)(
        u_c, u_a, u_t, ei_c, ei_a, ei_t, zb_h)

    def tc3(sp_ref, uc_ref, ua_ref, ut_ref, dis_ref, bc, ba, bt, l1w, l1b,
            w2, u2_o):
        sp = sp_ref[...]
        S = sp[:, 0] + sp[:, 1]
        dis = dis_ref[...]
        h = (dis[0] * (S[0] + uc_ref[...]) + bc[...]
             + dis[1] * (S[1] + ua_ref[...]) + ba[...]
             + dis[2] * (S[2] + ut_ref[...]) + bt[...])
        h = _elu(_layernorm(h, l1w[...], l1b[...]))
        u2_o[...] = jnp.dot(h, w2[...], preferred_element_type=f32) * dis[0]

    u2 = pl.pallas_call(
        tc3,
        grid=(n_blk,),
        in_specs=[rspec(3, 2, N, H)] + [rspec(N, H)] * 3
        + [rspec(3, N, 1)] + [fspec(1, H)] * 5 + [fspec(H, H)],
        out_specs=rspec(N, H),
        out_shape=jax.ShapeDtypeStruct((N, H), f32),
    )(seg_p, u_c, u_a, u_t, dis, r2(b_contig), r2(b_alliance), r2(b_trade),
      r2(ln1_w), r2(ln1_b), W_conv2)

    seg2_p = _make_sc_scatter(1, 1, H, N, n_grp)(u2, ei_c, zb_h)

    def tc4(sp_ref, u2_ref, dis_ref, b2, l2w, l2b, t_ref, expo_ref, wf1h,
            wf1t, wf1e, bf1_, wf2, bf2_, yf_o):
        sp = sp_ref[...]
        S2 = sp[0, 0] + sp[0, 1]
        dis = dis_ref[...]
        g = dis[0] * (S2 + u2_ref[...]) + b2[...]
        ho = _elu(_layernorm(g, l2w[...], l2b[...]))
        dot = lambda a, b: jnp.dot(a, b, preferred_element_type=f32)
        z = _elu(dot(ho, wf1h[...]) + dot(t_ref[...], wf1t[...])
                 + dot(expo_ref[...], wf1e[...]) + bf1_[...])
        yf_o[...] = _softmax(dot(z, wf2[...]) + bf2_[...])

    y_full = pl.pallas_call(
        tc4,
        grid=(n_blk,),
        in_specs=[rspec(1, 2, N, H), rspec(N, H), rspec(3, N, 1),
                  fspec(1, H), fspec(1, H), fspec(1, H),
                  rspec(N, T),
                  rspec(N, 3 * T), fspec(H, H), fspec(T, H),
                  fspec(3 * T, H), fspec(1, H), fspec(H, O), fspec(1, O)],
        out_specs=rspec(N, O),
        out_shape=jax.ShapeDtypeStruct((N, O), f32),
    )(seg2_p, u2, dis, r2(b_conv2), r2(ln2_w), r2(ln2_b), t_arr, expo,
      Wf1[:H], Wf1[H:H + T], Wf1[H + T:], r2(bf1), Wf2, r2(bf2))

    return (y_full, y_local, expo)
